# baseline (speedup 1.0000x reference)
import functools

import jax
import jax.numpy as jnp
from jax import lax
from jax.experimental import pallas as pl
from jax.experimental.pallas import tpu as pltpu
from jax.experimental.pallas import tpu_sc as plsc

_CHUNK = 256


@functools.lru_cache(maxsize=None)
def _make_sc_gather_exp(vocab: int, batch: int):
    info = plsc.get_sparse_core_info()
    nc, ns, nl = 1, info.num_subcores, info.num_lanes
    nw = nc * ns
    assert batch % (8 * nw) == 0
    b_per_w = batch // nw
    n_chunks = b_per_w // _CHUNK
    assert n_chunks * _CHUNK == b_per_w
    mesh = plsc.VectorSubcoreMesh(
        core_axis_name="c", subcore_axis_name="s", num_cores=1
    )

    @functools.partial(
        pl.kernel,
        mesh=mesh,
        out_type=jax.ShapeDtypeStruct((batch,), jnp.float32),
        scratch_types=[
            pltpu.VMEM((b_per_w,), jnp.int32),
            pltpu.VMEM((b_per_w,), jnp.float32),
            pltpu.SemaphoreType.DMA((3, n_chunks)),
        ],
    )
    def k(idx_hbm, table_hbm, out_hbm, idx_v, rows_v, sems):
        wid = lax.axis_index("s") * nc + lax.axis_index("c")
        base = wid * b_per_w
        idx_cp, gat_cp, out_cp = [], [], []
        for j in range(n_chunks):
            sl = pl.ds(j * _CHUNK, _CHUNK)
            idx_cp.append(
                pltpu.async_copy(
                    idx_hbm.at[pl.ds(base + j * _CHUNK, _CHUNK)],
                    idx_v.at[sl],
                    sems.at[0, j],
                )
            )
        for j in range(n_chunks):
            sl = pl.ds(j * _CHUNK, _CHUNK)
            idx_cp[j].wait()
            gat_cp.append(
                pltpu.async_copy(
                    table_hbm.at[idx_v.at[sl]], rows_v.at[sl], sems.at[1, j]
                )
            )
        for j in range(n_chunks):
            gat_cp[j].wait()

            def exp_body(i, _, j=j):
                sl = pl.ds(pl.multiple_of(j * _CHUNK + i * nl, nl), nl)
                rows_v[sl] = jnp.exp(rows_v[sl])
                return 0

            lax.fori_loop(0, _CHUNK // nl, exp_body, 0)
            sl = pl.ds(j * _CHUNK, _CHUNK)
            out_cp.append(
                pltpu.async_copy(
                    rows_v.at[sl],
                    out_hbm.at[pl.ds(base + j * _CHUNK, _CHUNK)],
                    sems.at[2, j],
                )
            )
        for c in out_cp:
            c.wait()

    return k


def kernel(x, temps):
    batch = x.shape[0]
    vocab = temps.shape[0]
    table = temps.reshape(vocab)
    out = _make_sc_gather_exp(vocab, batch)(x.astype(jnp.int32), table)
    return out.reshape(batch, 1)


# speedup vs baseline: 1.0211x; 1.0211x over previous
import functools

import jax
import jax.numpy as jnp
from jax import lax
from jax.experimental import pallas as pl
from jax.experimental.pallas import tpu as pltpu
from jax.experimental.pallas import tpu_sc as plsc

_CHUNK = 256


@functools.lru_cache(maxsize=None)
def _make_sc_gather_exp(vocab: int, batch: int):
    info = plsc.get_sparse_core_info()
    nc, ns, nl = 1, info.num_subcores, info.num_lanes
    nw = nc * ns
    assert batch % (8 * nw) == 0
    b_per_w = batch // nw
    n_chunks = b_per_w // _CHUNK
    assert n_chunks * _CHUNK == b_per_w
    mesh = plsc.VectorSubcoreMesh(
        core_axis_name="c", subcore_axis_name="s", num_cores=1
    )

    @functools.partial(
        pl.kernel,
        mesh=mesh,
        out_type=jax.ShapeDtypeStruct((batch,), jnp.float32),
        scratch_types=[
            pltpu.VMEM((b_per_w,), jnp.int32),
            pltpu.VMEM((b_per_w,), jnp.float32),
            pltpu.SemaphoreType.DMA((3, n_chunks)),
        ],
    )
    def k(idx_hbm, table_hbm, out_hbm, idx_v, rows_v, sems):
        wid = lax.axis_index("s") * nc + lax.axis_index("c")
        base = wid * b_per_w
        idx_cp, gat_cp, out_cp = [], [], []
        for j in range(n_chunks):
            sl = pl.ds(j * _CHUNK, _CHUNK)
            idx_cp.append(
                pltpu.async_copy(
                    idx_hbm.at[pl.ds(base + j * _CHUNK, _CHUNK)],
                    idx_v.at[sl],
                    sems.at[0, j],
                )
            )
        for j in range(n_chunks):
            sl = pl.ds(j * _CHUNK, _CHUNK)
            idx_cp[j].wait()
            gat_cp.append(
                pltpu.async_copy(
                    table_hbm.at[idx_v.at[sl]], rows_v.at[sl], sems.at[1, j]
                )
            )
        for j in range(n_chunks):
            gat_cp[j].wait()
            for i in range(_CHUNK // nl):
                sl = pl.ds(j * _CHUNK + i * nl, nl)
                rows_v[sl] = jnp.exp(rows_v[sl])
            sl = pl.ds(j * _CHUNK, _CHUNK)
            out_cp.append(
                pltpu.async_copy(
                    rows_v.at[sl],
                    out_hbm.at[pl.ds(base + j * _CHUNK, _CHUNK)],
                    sems.at[2, j],
                )
            )
        for c in out_cp:
            c.wait()

    return k


def kernel(x, temps):
    batch = x.shape[0]
    vocab = temps.shape[0]
    table = temps.reshape(vocab)
    out = _make_sc_gather_exp(vocab, batch)(x.astype(jnp.int32), table)
    return out.reshape(batch, 1)
